# baseline (speedup 1.0000x reference)
import functools

import numpy as np
import jax
import jax.numpy as jnp
from jax import lax
from jax.experimental import pallas as pl
from jax.experimental.pallas import tpu as pltpu
from jax.experimental.pallas import tpu_sc as plsc

_H = 8
_DH = 16

_NC = 2
_NS = 16
_L = 16
_C = 64
_S = 8


def _qkv_tc(node_feats, Wq, bq, Wk, bk, Wv, bv, block_n=1000):
    N, D = node_feats.shape
    M = Wq.shape[1]
    grid = N // block_n

    def body(x_ref, wq_ref, bq_ref, wk_ref, bk_ref, wv_ref, bv_ref,
             q_ref, k_ref, v_ref):
        x = x_ref[...]
        q_ref[...] = jnp.dot(x, wq_ref[...],
                             preferred_element_type=jnp.float32) + bq_ref[...]
        k_ref[...] = jnp.dot(x, wk_ref[...],
                             preferred_element_type=jnp.float32) + bk_ref[...]
        v_ref[...] = jnp.dot(x, wv_ref[...],
                             preferred_element_type=jnp.float32) + bv_ref[...]

    w_spec = pl.BlockSpec((D, M), lambda i: (0, 0))
    b_spec = pl.BlockSpec((1, M), lambda i: (0, 0))
    x_spec = pl.BlockSpec((block_n, D), lambda i: (i, 0))
    o_spec = pl.BlockSpec((block_n, M), lambda i: (i, 0))
    return pl.pallas_call(
        body,
        grid=(grid,),
        in_specs=[x_spec, w_spec, b_spec, w_spec, b_spec, w_spec, b_spec],
        out_specs=[o_spec, o_spec, o_spec],
        out_shape=[jax.ShapeDtypeStruct((N, M), jnp.float32)] * 3,
    )(node_feats, Wq, bq.reshape(1, M), Wk, bk.reshape(1, M), Wv,
      bv.reshape(1, M))


def _edge_sc(q, k, v, packed):
    N, D = q.shape
    NC, NS, L, C, S = _NC, _NS, _L, _C, _S
    E = packed.shape[0] // 2

    NW = NC * NS
    assert E % (NW * C * S) == 0 and C % L == 0 and C % 8 == 0
    ept = E // NW
    n_chunks = ept // C
    n_super = n_chunks // S
    NP = -(-N // (NS * 8)) * (NS * 8)
    rows_pt = NP // NS
    NB = NP // L

    mesh = plsc.VectorSubcoreMesh(core_axis_name="c", subcore_axis_name="s",
                                  num_cores=NC, num_subcores=NS)
    scale = float(1.0 / np.sqrt(_DH))

    @functools.partial(
        pl.kernel,
        out_type=[
            jax.ShapeDtypeStruct((NC, NP, D), jnp.float32),
            jax.ShapeDtypeStruct((NC, NB, D), jnp.float32),
        ],
        mesh=mesh,
        compiler_params=pltpu.CompilerParams(needs_layout_passes=False),
        scratch_types=[
            pltpu.VMEM((S * 2 * C,), jnp.int32),
            pltpu.VMEM((C,), jnp.int32),
            pltpu.VMEM((C,), jnp.int32),
            pltpu.VMEM((C,), jnp.int32),
            pltpu.VMEM((C, D), jnp.float32),
            pltpu.VMEM((C, D), jnp.float32),
            pltpu.VMEM((C, D), jnp.float32),
            pltpu.VMEM((C, D), jnp.float32),
            pltpu.VMEM_SHARED((NP, D), jnp.float32),
            pltpu.VMEM_SHARED((NB, D), jnp.float32),
            pltpu.SemaphoreType.DMA,
            pltpu.SemaphoreType.DMA,
        ],
    )
    def ker(q_hbm, k_hbm, v_hbm, packed_hbm, wv_out, z_out,
            pbuf, sbuf, dbuf, d2buf, kbuf, qbuf, vbuf, zrbuf,
            acc_wv, acc_z, sem, sem2):
        c = lax.axis_index("c")
        s = lax.axis_index("s")
        iota = lax.broadcasted_iota(jnp.int32, (L,), 0)
        r0 = pl.multiple_of(s * rows_pt, 8)

        def zb(i, carry):
            for j in range(D // L):
                vbuf[i, pl.ds(j * L, L)] = jnp.zeros((L,), jnp.float32)
                zrbuf[i, pl.ds(j * L, L)] = jnp.zeros((L,), jnp.float32)
            return carry
        lax.fori_loop(0, C, zb, 0)

        n_ch = -(-rows_pt // C)
        starts = [min(j * C, rows_pt - C) for j in range(n_ch)]
        zstride = -(-(-(-(NB - C) // (NS - 1))) // 8) * 8
        zst = pl.multiple_of(jnp.minimum(s * zstride, NB - C), 8)

        def fill_idx(row0):
            for g in range(C // L):
                sbuf[pl.ds(g * L, L)] = iota + (row0 + g * L)

        for st in starts:
            fill_idx(r0 + st)
            pltpu.sync_copy(vbuf, acc_wv.at[sbuf])
        fill_idx(zst)
        pltpu.sync_copy(zrbuf, acc_z.at[sbuf])

        plsc.subcore_barrier()

        chunk0 = (c * NS + s) * n_chunks

        def superstep(si, carry):
            soff = pl.multiple_of((chunk0 + si * S) * 2 * C, 8)
            pltpu.sync_copy(packed_hbm.at[pl.ds(soff, S * 2 * C)], pbuf)

            def chunk(cj, ccarry):
                off = cj * 2 * C
                for g in range(C // L):
                    sbuf[pl.ds(g * L, L)] = pbuf[pl.ds(off + g * L, L)]
                    dv = pbuf[pl.ds(off + C + g * L, L)]
                    dbuf[pl.ds(g * L, L)] = dv
                    d2buf[pl.ds(g * L, L)] = lax.shift_right_logical(dv, 4)
                cp_k = pltpu.async_copy(k_hbm.at[sbuf], kbuf, sem)
                cp_q = pltpu.async_copy(q_hbm.at[dbuf], qbuf, sem)
                cp_v = pltpu.async_copy(v_hbm.at[sbuf], vbuf, sem)
                cp_k.wait()
                cp_q.wait()
                cp_v.wait()

                def fused_group(g, gcarry):
                    e_vec = g * L + iota
                    dv = plsc.load_gather(dbuf, [e_vec])
                    colb = (dv & 15) * _H
                    for h in range(_H):
                        acc_s = None
                        for j in range(_DH):
                            d = jnp.full((L,), h * _DH + j, jnp.int32)
                            kv = plsc.load_gather(kbuf, [e_vec, d])
                            qv = plsc.load_gather(qbuf, [e_vec, d])
                            p = kv * qv
                            acc_s = p if acc_s is None else acc_s + p
                        sc = jnp.exp(jnp.clip(acc_s * scale, -5.0, 5.0))
                        plsc.store_scatter(zrbuf, [e_vec, colb + h], sc)
                        for j in range(_DH):
                            d = jnp.full((L,), h * _DH + j, jnp.int32)
                            vv = plsc.load_gather(vbuf, [e_vec, d])
                            plsc.store_scatter(kbuf, [e_vec, d], vv * sc)
                    return gcarry

                lax.fori_loop(0, C // L, fused_group, 0)

                cp_a = pltpu.async_copy(kbuf, acc_wv.at[dbuf], sem2,
                                        add=True)
                cp_b = pltpu.async_copy(zrbuf, acc_z.at[d2buf], sem2,
                                        add=True)
                cp_a.wait()
                cp_b.wait()

                def erase_group(g, gcarry):
                    e_vec = g * L + iota
                    dv = plsc.load_gather(dbuf, [e_vec])
                    colb = (dv & 15) * _H
                    zero = jnp.zeros((L,), jnp.float32)
                    for h in range(_H):
                        plsc.store_scatter(zrbuf, [e_vec, colb + h], zero)
                    return gcarry

                lax.fori_loop(0, C // L, erase_group, 0)
                return ccarry

            lax.fori_loop(0, S, chunk, 0)
            return carry

        lax.fori_loop(0, n_super, superstep, 0)

        plsc.subcore_barrier()

        for st in starts:
            row0 = pl.multiple_of(r0 + st, 8)
            fill_idx(row0)
            pltpu.async_copy(acc_wv.at[sbuf], vbuf, sem).wait()
            pltpu.sync_copy(vbuf, wv_out.at[c, pl.ds(row0, C)])
        fill_idx(zst)
        pltpu.async_copy(acc_z.at[sbuf], zrbuf, sem).wait()
        pltpu.sync_copy(zrbuf, z_out.at[c, pl.ds(zst, C)])

    return ker(q, k, v, packed)


def _finalize_tc(wv_p, z_p, block_n):
    NC, N, D = wv_p.shape
    L = z_p.shape[2]
    grid = N // block_n

    def body(wv_ref, z_ref, o_ref):
        wv = wv_ref[0] + wv_ref[1]
        z = z_ref[0] + z_ref[1]
        hsel = lax.broadcasted_iota(jnp.int32, (L, D), 0)
        dsel = lax.broadcasted_iota(jnp.int32, (L, D), 1) // _DH
        sel = jnp.where(hsel == dsel, 1.0, 0.0)
        zfull = jnp.dot(z, sel, preferred_element_type=jnp.float32)
        o_ref[...] = wv / (zfull + 1e-6)

    return pl.pallas_call(
        body,
        grid=(grid,),
        in_specs=[
            pl.BlockSpec((NC, block_n, D), lambda i: (0, i, 0)),
            pl.BlockSpec((NC, block_n, L), lambda i: (0, i, 0)),
        ],
        out_specs=pl.BlockSpec((block_n, D), lambda i: (i, 0)),
        out_shape=jax.ShapeDtypeStruct((N, D), jnp.float32),
    )(wv_p, z_p)


def kernel(node_feats, edge_index, Wq, bq, Wk, bk, Wv, bv):
    N, D = node_feats.shape
    E = edge_index.shape[1]
    q, k, v = _qkv_tc(node_feats, Wq, bq, Wk, bk, Wv, bv)
    NP = -(-N // (_NS * 8)) * (_NS * 8)
    rows_pt = NP // _NS
    unit = _NC * _NS * _C * _S
    EP = -(-E // unit) * unit
    src = edge_index[0]
    dst = edge_index[1]
    if EP != E:
        pad = EP - E
        src = jnp.concatenate([src, jnp.zeros((pad,), jnp.int32)])
        dst = jnp.concatenate([dst, jnp.full((pad,), N, jnp.int32)])
    packed = jnp.stack(
        [src.reshape(-1, _C), dst.reshape(-1, _C)], axis=1).reshape(-1)
    wv_p, z_b = _edge_sc(q, k, v, packed)
    z_p = z_b.reshape(_NC, NP, _H)
    out = _finalize_tc(wv_p, z_p, block_n=rows_pt)
    return out[:N].reshape(N, _H, _DH)


# speedup vs baseline: 1.0847x; 1.0847x over previous
import functools

import numpy as np
import jax
import jax.numpy as jnp
from jax import lax
from jax.experimental import pallas as pl
from jax.experimental.pallas import tpu as pltpu
from jax.experimental.pallas import tpu_sc as plsc

_H = 8
_DH = 16

_NC = 2
_NS = 16
_L = 16
_C = 64
_S = 8


def _skew(a, sign):
    m = lax.broadcasted_iota(jnp.int32, a.shape, 0) % _L
    res = a
    for r in range(1, _L):
        res = jnp.where(m == r, jnp.roll(a, sign * 8 * r, axis=1), res)
    return res


def _qkv_tc(node_feats, Wq, bq, Wk, bk, Wv, bv, block_n=2000):
    N, D = node_feats.shape
    M = Wq.shape[1]
    assert block_n % _L == 0
    grid = N // block_n

    def body(x_ref, wq_ref, bq_ref, wk_ref, bk_ref, wv_ref, bv_ref,
             q_ref, k_ref, v_ref):
        x = x_ref[...]
        q_ref[...] = _skew(jnp.dot(x, wq_ref[...],
                                   preferred_element_type=jnp.float32)
                           + bq_ref[...], 1)
        k_ref[...] = _skew(jnp.dot(x, wk_ref[...],
                                   preferred_element_type=jnp.float32)
                           + bk_ref[...], 1)
        v_ref[...] = _skew(jnp.dot(x, wv_ref[...],
                                   preferred_element_type=jnp.float32)
                           + bv_ref[...], 1)

    w_spec = pl.BlockSpec((D, M), lambda i: (0, 0))
    b_spec = pl.BlockSpec((1, M), lambda i: (0, 0))
    x_spec = pl.BlockSpec((block_n, D), lambda i: (i, 0))
    o_spec = pl.BlockSpec((block_n, M), lambda i: (i, 0))
    return pl.pallas_call(
        body,
        grid=(grid,),
        in_specs=[x_spec, w_spec, b_spec, w_spec, b_spec, w_spec, b_spec],
        out_specs=[o_spec, o_spec, o_spec],
        out_shape=[jax.ShapeDtypeStruct((N, M), jnp.float32)] * 3,
    )(node_feats, Wq, bq.reshape(1, M), Wk, bk.reshape(1, M), Wv,
      bv.reshape(1, M))


def _edge_sc(q, k, v, packed):
    N, D = q.shape
    NC, NS, L, C, S = _NC, _NS, _L, _C, _S
    E = packed.shape[0] // 2

    NW = NC * NS
    assert E % (NW * C * S) == 0 and C % L == 0 and C % 8 == 0
    ept = E // NW
    n_chunks = ept // C
    n_super = n_chunks // S
    NP = -(-N // (NS * 8)) * (NS * 8)
    rows_pt = NP // NS
    NB = NP // L

    mesh = plsc.VectorSubcoreMesh(core_axis_name="c", subcore_axis_name="s",
                                  num_cores=NC, num_subcores=NS)
    scale = float(1.0 / np.sqrt(_DH))

    @functools.partial(
        pl.kernel,
        out_type=[
            jax.ShapeDtypeStruct((NC, NP, D), jnp.float32),
            jax.ShapeDtypeStruct((NC, NB, D), jnp.float32),
        ],
        mesh=mesh,
        compiler_params=pltpu.CompilerParams(needs_layout_passes=False),
        scratch_types=[
            pltpu.VMEM((S * 2 * C,), jnp.int32),
            pltpu.VMEM((C,), jnp.int32),
            pltpu.VMEM((C,), jnp.int32),
            pltpu.VMEM((C,), jnp.int32),
            pltpu.VMEM((C, D), jnp.float32),
            pltpu.VMEM((C, D), jnp.float32),
            pltpu.VMEM((C, D), jnp.float32),
            pltpu.VMEM((C, D), jnp.float32),
            pltpu.VMEM_SHARED((NP, D), jnp.float32),
            pltpu.VMEM_SHARED((NB, D), jnp.float32),
            pltpu.SemaphoreType.DMA,
            pltpu.SemaphoreType.DMA,
        ],
    )
    def ker(q_hbm, k_hbm, v_hbm, packed_hbm, wv_out, z_out,
            pbuf, sbuf, dbuf, d2buf, kbuf, qbuf, vbuf, zrbuf,
            acc_wv, acc_z, sem, sem2):
        c = lax.axis_index("c")
        s = lax.axis_index("s")
        iota = lax.broadcasted_iota(jnp.int32, (L,), 0)
        r0 = pl.multiple_of(s * rows_pt, 8)

        def zb(i, carry):
            for j in range(D // L):
                vbuf[i, pl.ds(j * L, L)] = jnp.zeros((L,), jnp.float32)
                zrbuf[i, pl.ds(j * L, L)] = jnp.zeros((L,), jnp.float32)
            return carry
        lax.fori_loop(0, C, zb, 0)

        n_ch = -(-rows_pt // C)
        starts = [min(j * C, rows_pt - C) for j in range(n_ch)]
        zstride = -(-(-(-(NB - C) // (NS - 1))) // 8) * 8
        zst = pl.multiple_of(jnp.minimum(s * zstride, NB - C), 8)

        def fill_idx(row0):
            for g in range(C // L):
                sbuf[pl.ds(g * L, L)] = iota + (row0 + g * L)

        for st in starts:
            fill_idx(r0 + st)
            pltpu.sync_copy(vbuf, acc_wv.at[sbuf])
        fill_idx(zst)
        pltpu.sync_copy(zrbuf, acc_z.at[sbuf])

        plsc.subcore_barrier()

        chunk0 = (c * NS + s) * n_chunks

        def superstep(si, carry):
            soff = pl.multiple_of((chunk0 + si * S) * 2 * C, 8)
            pltpu.sync_copy(packed_hbm.at[pl.ds(soff, S * 2 * C)], pbuf)

            def chunk(cj, ccarry):
                off = cj * 2 * C
                for g in range(C // L):
                    sbuf[pl.ds(g * L, L)] = pbuf[pl.ds(off + g * L, L)]
                    dv = pbuf[pl.ds(off + C + g * L, L)]
                    dbuf[pl.ds(g * L, L)] = dv
                    d2buf[pl.ds(g * L, L)] = lax.shift_right_logical(dv, 4)
                cp_k = pltpu.async_copy(k_hbm.at[sbuf], kbuf, sem)
                cp_q = pltpu.async_copy(q_hbm.at[dbuf], qbuf, sem)
                cp_v = pltpu.async_copy(v_hbm.at[sbuf], vbuf, sem)
                cp_k.wait()
                cp_q.wait()
                cp_v.wait()

                def score_group(g, gcarry):
                    e_vec = g * L + iota
                    sv = plsc.load_gather(sbuf, [e_vec])
                    dv = plsc.load_gather(dbuf, [e_vec])
                    ks = (sv & 15) * 8
                    qs = (dv & 15) * 8
                    colb = (dv & 15) * _H
                    for h in range(_H):
                        acc_s = None
                        for j in range(_DH):
                            d0 = h * _DH + j
                            kv = plsc.load_gather(kbuf, [e_vec,
                                                         (ks + d0) & 127])
                            qv = plsc.load_gather(qbuf, [e_vec,
                                                         (qs + d0) & 127])
                            p = kv * qv
                            acc_s = p if acc_s is None else acc_s + p
                        sc = jnp.exp(jnp.clip(acc_s * scale, -5.0, 5.0))
                        plsc.store_scatter(zrbuf, [e_vec, colb + h], sc)
                    return gcarry

                lax.fori_loop(0, C // L, score_group, 0)

                def weight_group(g, gcarry):
                    e_vec = g * L + iota
                    sv = plsc.load_gather(sbuf, [e_vec])
                    dv = plsc.load_gather(dbuf, [e_vec])
                    ks = (sv & 15) * 8
                    qs = (dv & 15) * 8
                    colb = (dv & 15) * _H
                    for h in range(_H):
                        sc = plsc.load_gather(zrbuf, [e_vec, colb + h])
                        for j in range(_DH):
                            d0 = h * _DH + j
                            vv = plsc.load_gather(vbuf, [e_vec,
                                                         (ks + d0) & 127])
                            plsc.store_scatter(qbuf, [e_vec,
                                                      (qs + d0) & 127],
                                               vv * sc)
                    return gcarry

                lax.fori_loop(0, C // L, weight_group, 0)

                cp_a = pltpu.async_copy(qbuf, acc_wv.at[dbuf], sem2,
                                        add=True)
                cp_b = pltpu.async_copy(zrbuf, acc_z.at[d2buf], sem2,
                                        add=True)
                cp_a.wait()
                cp_b.wait()

                def erase_group(g, gcarry):
                    e_vec = g * L + iota
                    dv = plsc.load_gather(dbuf, [e_vec])
                    colb = (dv & 15) * _H
                    zero = jnp.zeros((L,), jnp.float32)
                    for h in range(_H):
                        plsc.store_scatter(zrbuf, [e_vec, colb + h], zero)
                    return gcarry

                lax.fori_loop(0, C // L, erase_group, 0)
                return ccarry

            lax.fori_loop(0, S, chunk, 0)
            return carry

        lax.fori_loop(0, n_super, superstep, 0)

        plsc.subcore_barrier()

        for st in starts:
            row0 = pl.multiple_of(r0 + st, 8)
            fill_idx(row0)
            pltpu.async_copy(acc_wv.at[sbuf], vbuf, sem).wait()
            pltpu.sync_copy(vbuf, wv_out.at[c, pl.ds(row0, C)])
        fill_idx(zst)
        pltpu.async_copy(acc_z.at[sbuf], zrbuf, sem).wait()
        pltpu.sync_copy(zrbuf, z_out.at[c, pl.ds(zst, C)])

    return ker(q, k, v, packed)


def _finalize_tc(wv_p, z_p, block_n):
    NC, N, D = wv_p.shape
    L = z_p.shape[2]
    grid = N // block_n

    def body(wv_ref, z_ref, o_ref):
        wv = _skew(wv_ref[0] + wv_ref[1], -1)
        z = z_ref[0] + z_ref[1]
        hsel = lax.broadcasted_iota(jnp.int32, (L, D), 0)
        dsel = lax.broadcasted_iota(jnp.int32, (L, D), 1) // _DH
        sel = jnp.where(hsel == dsel, 1.0, 0.0)
        zfull = jnp.dot(z, sel, preferred_element_type=jnp.float32)
        o_ref[...] = wv / (zfull + 1e-6)

    return pl.pallas_call(
        body,
        grid=(grid,),
        in_specs=[
            pl.BlockSpec((NC, block_n, D), lambda i: (0, i, 0)),
            pl.BlockSpec((NC, block_n, L), lambda i: (0, i, 0)),
        ],
        out_specs=pl.BlockSpec((block_n, D), lambda i: (i, 0)),
        out_shape=jax.ShapeDtypeStruct((N, D), jnp.float32),
    )(wv_p, z_p)


def kernel(node_feats, edge_index, Wq, bq, Wk, bk, Wv, bv):
    N, D = node_feats.shape
    E = edge_index.shape[1]
    q, k, v = _qkv_tc(node_feats, Wq, bq, Wk, bk, Wv, bv)
    NP = -(-N // (_NS * 8)) * (_NS * 8)
    rows_pt = NP // _NS
    unit = _NC * _NS * _C * _S
    EP = -(-E // unit) * unit
    src = edge_index[0]
    dst = edge_index[1]
    if EP != E:
        pad = EP - E
        src = jnp.concatenate([src, jnp.zeros((pad,), jnp.int32)])
        dst = jnp.concatenate([dst, jnp.full((pad,), N, jnp.int32)])
    packed = jnp.stack(
        [src.reshape(-1, _C), dst.reshape(-1, _C)], axis=1).reshape(-1)
    wv_p, z_b = _edge_sc(q, k, v, packed)
    z_p = z_b.reshape(_NC, NP, _H)
    out = _finalize_tc(wv_p, z_p, block_n=2 * rows_pt)
    return out[:N].reshape(N, _H, _DH)
